# baseline (speedup 1.0000x reference)
import jax
import jax.numpy as jnp
from jax.experimental import pallas as pl
from jax.experimental.pallas import tpu as pltpu


def _pick_tm(n):
    for tm in (400, 200, 100, 50, 25, 10, 8, 5, 4, 2):
        if n % tm == 0:
            return tm
    return 1


def _gcn_body(feat_ref, w_ref, adj_ref, bias_ref, out_ref, fw_ref):
    step = pl.program_id(0)

    @pl.when(step == 0)
    def _():
        fw = jnp.dot(feat_ref[...], w_ref[...],
                     preferred_element_type=jnp.float32)
        fw_ref[...] = fw

    acc = jnp.dot(adj_ref[...], fw_ref[...],
                  preferred_element_type=jnp.float32)
    out_ref[...] = jnp.maximum(acc + bias_ref[...], 0.0)


def kernel(adj, feature, weight, bias):
    n, d = feature.shape
    f = weight.shape[1]
    tm = _pick_tm(n)
    grid = (n // tm,)
    return pl.pallas_call(
        _gcn_body,
        grid=grid,
        in_specs=[
            pl.BlockSpec((n, d), lambda i: (0, 0)),
            pl.BlockSpec((d, f), lambda i: (0, 0)),
            pl.BlockSpec((tm, n), lambda i: (i, 0)),
            pl.BlockSpec((tm, f), lambda i: (i, 0)),
        ],
        out_specs=pl.BlockSpec((tm, f), lambda i: (i, 0)),
        out_shape=jax.ShapeDtypeStruct((n, f), jnp.float32),
        scratch_shapes=[pltpu.VMEM((n, f), jnp.float32)],
    )(feature, weight, adj, bias)


# speedup vs baseline: 1.0180x; 1.0180x over previous
import jax
import jax.numpy as jnp
from jax.experimental import pallas as pl
from jax.experimental.pallas import tpu as pltpu


def _pick_tm(n):
    for tm in (200, 100, 50, 25, 10, 8, 5, 4, 2):
        if n % tm == 0:
            return tm
    return 1


def _gcn_body(feat_ref, w_ref, adj_ref, bias_ref, out_ref, fw_ref):
    step = pl.program_id(0)

    @pl.when(step == 0)
    def _():
        fw = jnp.dot(feat_ref[...], w_ref[...],
                     preferred_element_type=jnp.float32)
        fw_ref[...] = fw

    acc = jnp.dot(adj_ref[...], fw_ref[...],
                  preferred_element_type=jnp.float32)
    out_ref[...] = jnp.maximum(acc + bias_ref[...], 0.0)


def kernel(adj, feature, weight, bias):
    n, d = feature.shape
    f = weight.shape[1]
    tm = _pick_tm(n)
    grid = (n // tm,)
    return pl.pallas_call(
        _gcn_body,
        grid=grid,
        in_specs=[
            pl.BlockSpec((n, d), lambda i: (0, 0)),
            pl.BlockSpec((d, f), lambda i: (0, 0)),
            pl.BlockSpec((tm, n), lambda i: (i, 0)),
            pl.BlockSpec((tm, f), lambda i: (i, 0)),
        ],
        out_specs=pl.BlockSpec((tm, f), lambda i: (i, 0)),
        out_shape=jax.ShapeDtypeStruct((n, f), jnp.float32),
        scratch_shapes=[pltpu.VMEM((n, f), jnp.float32)],
    )(feature, weight, adj, bias)
